# baseline (device time: 41560 ns/iter reference)
import os

import jax
import jax.numpy as jnp
from jax import lax
from jax.experimental import pallas as pl
from jax.experimental.pallas import tpu as pltpu

_KVAR = os.environ.get("KVAR", "full")

N_DEV = 8
B = 64
D = 512
H = 1024
HC = H // N_DEV


def kernel(x, Win0, Wout0, Win1, Wout1, Win2, Wout2):
    if _KVAR == "empty":
        def _copy_body(x_ref, *refs):
            refs[-1][:, :] = x_ref[:, :]

        return pl.pallas_call(
            _copy_body,
            out_shape=jax.ShapeDtypeStruct((B, D), jnp.float32),
            in_specs=[pl.BlockSpec(memory_space=pltpu.VMEM)] * 7,
            out_specs=pl.BlockSpec(memory_space=pltpu.VMEM),
        )(x, Win0, Wout0, Win1, Wout1, Win2, Wout2)

    def body(x_ref, win0_ref, wout0_ref, win1_ref, wout1_ref, win2_ref,
             wout2_ref, out_ref, partial_ref, recva_ref, h_ref, red_ref,
             xv_ref, win0v_ref, wout0v_ref, win1v_ref, wout1v_ref,
             win2v_ref, wout2v_ref,
             senda_sems, recva_sems, sendb_sems, recvb_sems, load_sems):
        my = lax.axis_index("i")

        loads = []
        for i, (src, dst) in enumerate([
                (x_ref, xv_ref), (win0_ref, win0v_ref),
                (wout0_ref, wout0v_ref), (win1_ref, win1v_ref),
                (wout1_ref, wout1v_ref), (win2_ref, win2v_ref),
                (wout2_ref, wout2v_ref)]):
            cp = pltpu.make_async_copy(src, dst, load_sems.at[i])
            cp.start()
            loads.append(cp)
        slot_mask = lax.broadcasted_iota(jnp.int32, (N_DEV, 1, 1), 0) == my

        if _KVAR == "full":
            bar = pltpu.get_barrier_semaphore()
            for off in range(1, N_DEV):
                t = lax.rem(my + off, N_DEV)
                pl.semaphore_signal(bar, inc=1, device_id=(t,),
                                    device_id_type=pl.DeviceIdType.MESH)

        wins = [win0v_ref, win1v_ref, win2v_ref]
        wouts = [wout0v_ref, wout1v_ref, wout2v_ref]

        loads[0].wait()
        x_cur = xv_ref[:, :]
        for l in range(3):
            loads[1 + 2 * l].wait()
            partial = jnp.dot(x_cur, wins[l][:, :],
                              preferred_element_type=jnp.float32)
            p3 = jnp.swapaxes(partial.reshape(B, N_DEV, HC), 0, 1)
            partial_ref[:, :, :] = p3
            if _KVAR == "nocomm" and l == 0:
                recva_ref[:, :, :] = p3[: N_DEV - 1]
                h_ref[:, :, :] = p3
            if _KVAR == "full" and l == 0:
                pl.semaphore_wait(bar, N_DEV - 1)

            rdmas_a = []
            if _KVAR == "full":
                for off in range(1, N_DEV):
                    k = off - 1
                    t = lax.rem(my + off, N_DEV)
                    rdma = pltpu.make_async_remote_copy(
                        src_ref=partial_ref.at[t],
                        dst_ref=recva_ref.at[k],
                        send_sem=senda_sems.at[k],
                        recv_sem=recva_sems.at[k],
                        device_id=(t,),
                        device_id_type=pl.DeviceIdType.MESH,
                    )
                    rdma.start()
                    rdmas_a.append(rdma)
            acc = jnp.sum(jnp.where(slot_mask, p3, 0.0), axis=0)
            for k in range(N_DEV - 1):
                if rdmas_a:
                    rdmas_a[k].wait_recv()
                acc = acc + recva_ref[k]
            hred = jnp.maximum(acc, 0.0)
            red_ref[:, :] = hred

            rdmas_b = []
            if _KVAR == "full":
                for off in range(1, N_DEV):
                    k = off - 1
                    t = lax.rem(my + off, N_DEV)
                    rdma = pltpu.make_async_remote_copy(
                        src_ref=red_ref,
                        dst_ref=h_ref.at[my],
                        send_sem=sendb_sems.at[k],
                        recv_sem=recvb_sems.at[k],
                        device_id=(t,),
                        device_id_type=pl.DeviceIdType.MESH,
                    )
                    rdma.start()
                    rdmas_b.append(rdma)
            for r in rdmas_b:
                r.wait_recv()

            loads[2 + 2 * l].wait()
            h3 = jnp.where(slot_mask, hred[None, :, :], h_ref[:, :, :])
            h_full = jnp.swapaxes(h3, 0, 1).reshape(B, H)
            x_cur = jnp.dot(h_full, wouts[l][:, :],
                            preferred_element_type=jnp.float32)

            for r in rdmas_a:
                r.wait_send()
            for r in rdmas_b:
                r.wait_send()

        out_ref[:, :] = x_cur

    return pl.pallas_call(
        body,
        out_shape=jax.ShapeDtypeStruct((B, D), jnp.float32),
        in_specs=[pl.BlockSpec(memory_space=pltpu.MemorySpace.HBM)] * 7,
        out_specs=pl.BlockSpec(memory_space=pltpu.VMEM),
        scratch_shapes=[
            pltpu.VMEM((N_DEV, B, HC), jnp.float32),
            pltpu.VMEM((N_DEV - 1, B, HC), jnp.float32),
            pltpu.VMEM((N_DEV, B, HC), jnp.float32),
            pltpu.VMEM((B, HC), jnp.float32),
            pltpu.VMEM((B, D), jnp.float32),
            pltpu.VMEM((D, H), jnp.float32),
            pltpu.VMEM((H, D), jnp.float32),
            pltpu.VMEM((D, H), jnp.float32),
            pltpu.VMEM((H, D), jnp.float32),
            pltpu.VMEM((D, H), jnp.float32),
            pltpu.VMEM((H, D), jnp.float32),
            pltpu.SemaphoreType.DMA((N_DEV - 1,)),
            pltpu.SemaphoreType.DMA((N_DEV - 1,)),
            pltpu.SemaphoreType.DMA((N_DEV - 1,)),
            pltpu.SemaphoreType.DMA((N_DEV - 1,)),
            pltpu.SemaphoreType.DMA((7,)),
        ],
        compiler_params=(
            pltpu.CompilerParams(collective_id=0)
            if _KVAR == "full" else pltpu.CompilerParams()
        ),
    )(x, Win0, Wout0, Win1, Wout1, Win2, Wout2)


# device time: 36848 ns/iter; 1.1279x vs baseline; 1.1279x over previous
import os

import jax
import jax.numpy as jnp
from jax import lax
from jax.experimental import pallas as pl
from jax.experimental.pallas import tpu as pltpu

_KVAR = os.environ.get("KVAR", "full")

N_DEV = 8
B = 64
D = 512
H = 1024
HC = H // N_DEV


def kernel(x, Win0, Wout0, Win1, Wout1, Win2, Wout2):
    def body(x_ref, win0_ref, wout0_ref, win1_ref, wout1_ref, win2_ref,
             wout2_ref, out_ref, partial_ref, recva_ref, h_ref, red_ref,
             senda_sems, recva_sems, sendb_sems, recvb_sems, local_sem):
        my = lax.axis_index("i")
        slot_mask = lax.broadcasted_iota(jnp.int32, (N_DEV, 1, 1), 0) == my

        if _KVAR == "full":
            bar = pltpu.get_barrier_semaphore()
            for off in range(1, N_DEV):
                t = lax.rem(my + off, N_DEV)
                pl.semaphore_signal(bar, inc=1, device_id=(t,),
                                    device_id_type=pl.DeviceIdType.MESH)

        wins = [win0_ref, win1_ref, win2_ref]
        wouts = [wout0_ref, wout1_ref, wout2_ref]

        x_cur = x_ref[:, :]
        for l in range(3):
            partial = jnp.dot(x_cur, wins[l][:, :],
                              preferred_element_type=jnp.float32)
            p3f = jnp.swapaxes(partial.reshape(B, N_DEV, HC), 0, 1)
            p3 = p3f.astype(jnp.bfloat16)
            partial_ref[:, :, :] = p3
            if _KVAR == "nocomm" and l == 0:
                recva_ref[:, :, :] = p3[: N_DEV - 1]
                h_ref[:, :, :] = p3
            if _KVAR == "full" and l == 0:
                pl.semaphore_wait(bar, N_DEV - 1)

            rdmas_a = []
            if _KVAR == "full":
                for off in range(1, N_DEV):
                    k = off - 1
                    t = lax.rem(my + off, N_DEV)
                    rdma = pltpu.make_async_remote_copy(
                        src_ref=partial_ref.at[t],
                        dst_ref=recva_ref.at[k],
                        send_sem=senda_sems.at[k],
                        recv_sem=recva_sems.at[k],
                        device_id=(t,),
                        device_id_type=pl.DeviceIdType.MESH,
                    )
                    rdma.start()
                    rdmas_a.append(rdma)
            acc = jnp.sum(jnp.where(slot_mask, p3f, 0.0), axis=0)
            for k in range(N_DEV - 1):
                if rdmas_a:
                    rdmas_a[k].wait_recv()
                acc = acc + recva_ref[k].astype(jnp.float32)
            hred = jnp.maximum(acc, 0.0).astype(jnp.bfloat16)
            red_ref[:, :] = hred

            rdmas_b = []
            if _KVAR == "full":
                for off in range(1, N_DEV):
                    k = off - 1
                    t = lax.rem(my + off, N_DEV)
                    rdma = pltpu.make_async_remote_copy(
                        src_ref=red_ref,
                        dst_ref=h_ref.at[my],
                        send_sem=sendb_sems.at[k],
                        recv_sem=recvb_sems.at[k],
                        device_id=(t,),
                        device_id_type=pl.DeviceIdType.MESH,
                    )
                    rdma.start()
                    rdmas_b.append(rdma)
            for r in rdmas_b:
                r.wait_recv()

            h3 = jnp.where(slot_mask, hred[None, :, :], h_ref[:, :, :])
            h_full = jnp.swapaxes(h3, 0, 1).reshape(B, H)
            x_cur = jnp.dot(h_full, wouts[l][:, :],
                            preferred_element_type=jnp.float32)

            for r in rdmas_a:
                r.wait_send()
            for r in rdmas_b:
                r.wait_send()

        out_ref[:, :] = x_cur

    return pl.pallas_call(
        body,
        out_shape=jax.ShapeDtypeStruct((B, D), jnp.float32),
        in_specs=[pl.BlockSpec(memory_space=pltpu.VMEM)] * 7,
        out_specs=pl.BlockSpec(memory_space=pltpu.VMEM),
        scratch_shapes=[
            pltpu.VMEM((N_DEV, B, HC), jnp.bfloat16),
            pltpu.VMEM((N_DEV - 1, B, HC), jnp.bfloat16),
            pltpu.VMEM((N_DEV, B, HC), jnp.bfloat16),
            pltpu.VMEM((B, HC), jnp.bfloat16),
            pltpu.SemaphoreType.DMA((N_DEV - 1,)),
            pltpu.SemaphoreType.DMA((N_DEV - 1,)),
            pltpu.SemaphoreType.DMA((N_DEV - 1,)),
            pltpu.SemaphoreType.DMA((N_DEV - 1,)),
            pltpu.SemaphoreType.DMA,
        ],
        compiler_params=(
            pltpu.CompilerParams(collective_id=0)
            if _KVAR == "full" else pltpu.CompilerParams()
        ),
    )(x, Win0, Wout0, Win1, Wout1, Win2, Wout2)
